# baseline (speedup 1.0000x reference)
import functools

import jax
import jax.numpy as jnp
from jax import lax
from jax.experimental import pallas as pl
from jax.experimental.pallas import tpu as pltpu
from jax.experimental.pallas import tpu_sc as plsc

_VOCAB = 100000
_DIM = 64
_BATCH = 1024
_HB = _BATCH // 2
_TV = 4096
_NB = 8


def _gather_rows_sc(embt, idx, base):
    info = plsc.get_sparse_core_info()
    nc, ns, nl = info.num_cores, info.num_subcores, info.num_lanes
    nw = nc * ns
    bpw = _HB // nw
    mesh = plsc.VectorSubcoreMesh(core_axis_name="c", subcore_axis_name="s")

    @functools.partial(
        pl.kernel,
        mesh=mesh,
        compiler_params=pltpu.CompilerParams(needs_layout_passes=False),
        out_type=jax.ShapeDtypeStruct((_HB, _DIM), jnp.float32),
        scratch_types=[
            pltpu.VMEM((bpw,), jnp.int32),
            pltpu.VMEM((_DIM, _NB * 128), jnp.float32),
            pltpu.VMEM((bpw, _DIM), jnp.float32),
            pltpu.SemaphoreType.DMA((_NB,)),
        ],
    )
    def gk(embt_hbm, idx_hbm, out_hbm, idx_v, blk_v, rows_v, sems):
        wid = lax.axis_index("s") * nc + lax.axis_index("c")
        wbase = wid * bpw
        pltpu.sync_copy(idx_hbm.at[pl.ds(base + wbase, bpw)], idx_v)

        def token_col(i):
            group = idx_v[pl.ds(i - i % nl, nl)]
            sel = lax.iota(jnp.int32, nl) == (i % nl)
            return lax.reduce_max(jnp.where(sel, group, 0), (0,))

        def fire(i):
            col = token_col(i)
            col0 = pl.multiple_of((col // 128) * 128, 128)
            s = i % _NB
            return (
                pltpu.async_copy(
                    embt_hbm.at[:, pl.ds(col0, 128)],
                    blk_v.at[:, pl.ds(s * 128, 128)],
                    sems.at[s],
                ),
                col - col0,
            )

        ring = [fire(i) for i in range(_NB)]
        for i in range(bpw):
            desc, r = ring[i % _NB]
            desc.wait()
            lane = i % _NB * 128 + r
            for k in range(_DIM // nl):
                d = lax.iota(jnp.int32, nl) + k * nl
                vals = plsc.load_gather(blk_v, [d, jnp.full((nl,), 0, jnp.int32) + lane])
                plsc.store_scatter(
                    rows_v, [jnp.full((nl,), i, jnp.int32), d], vals
                )
            if i + _NB < bpw:
                ring[i % _NB] = fire(i + _NB)
        pltpu.sync_copy(rows_v, out_hbm.at[pl.ds(wbase, bpw)])

    return gk(embt, idx)


def _matmul_first(x_ref, w_ref, b_ref, out_ref):
    lhs = jnp.concatenate([w_ref[...], b_ref[...]], axis=0).astype(jnp.bfloat16)
    rhs = jnp.concatenate(
        [x_ref[...], jnp.ones((_HB, 1), jnp.float32)], axis=1
    ).astype(jnp.bfloat16)
    out_ref[...] = jax.lax.dot_general(
        lhs, rhs, (((0,), (1,)), ((), ())), preferred_element_type=jnp.float32
    )


def _matmul_second(prev_any, x_ref, w_ref, b_ref, out_ref):
    del prev_any
    _matmul_first(x_ref, w_ref, b_ref, out_ref)


def kernel(input_ids, emb, W, b):
    idx = input_ids.astype(jnp.int32)
    embt = jnp.swapaxes(emb, 0, 1)
    xa = _gather_rows_sc(embt, idx, 0)
    xb = _gather_rows_sc(embt, idx, _HB)
    b2 = b.reshape(1, _VOCAB)
    grid = (pl.cdiv(_VOCAB, _TV),)
    half_a = pl.pallas_call(
        _matmul_first,
        grid=grid,
        in_specs=[
            pl.BlockSpec((_HB, _DIM), lambda i: (0, 0)),
            pl.BlockSpec((_DIM, _TV), lambda i: (0, i)),
            pl.BlockSpec((1, _TV), lambda i: (0, i)),
        ],
        out_specs=pl.BlockSpec((_TV, _HB), lambda i: (i, 0)),
        out_shape=jax.ShapeDtypeStruct((_VOCAB, _BATCH), jnp.float32),
        compiler_params=pltpu.CompilerParams(
            dimension_semantics=("arbitrary",),
        ),
    )(xa, W, b2)
    logits_t = pl.pallas_call(
        _matmul_second,
        grid=grid,
        in_specs=[
            pl.BlockSpec(memory_space=pl.ANY),
            pl.BlockSpec((_HB, _DIM), lambda i: (0, 0)),
            pl.BlockSpec((_DIM, _TV), lambda i: (0, i)),
            pl.BlockSpec((1, _TV), lambda i: (0, i)),
        ],
        out_specs=pl.BlockSpec((_TV, _HB), lambda i: (i, 1)),
        out_shape=jax.ShapeDtypeStruct((_VOCAB, _BATCH), jnp.float32),
        input_output_aliases={0: 0},
        compiler_params=pltpu.CompilerParams(
            dimension_semantics=("arbitrary",),
        ),
    )(half_a, xb, W, b2)
    return jnp.transpose(logits_t)


# speedup vs baseline: 1.0709x; 1.0709x over previous
import functools

import jax
import jax.numpy as jnp
from jax import lax
from jax.experimental import pallas as pl
from jax.experimental.pallas import tpu as pltpu
from jax.experimental.pallas import tpu_sc as plsc

_VOCAB = 100000
_DIM = 64
_BATCH = 1024
_TV = 4096
_NB = 12


def _gather_rows_sc(embt, idx):
    info = plsc.get_sparse_core_info()
    nc, ns, nl = info.num_cores, info.num_subcores, info.num_lanes
    nw = nc * ns
    bpw = _BATCH // nw
    mesh = plsc.VectorSubcoreMesh(core_axis_name="c", subcore_axis_name="s")

    @functools.partial(
        pl.kernel,
        mesh=mesh,
        compiler_params=pltpu.CompilerParams(needs_layout_passes=False),
        out_type=jax.ShapeDtypeStruct((_BATCH, _DIM), jnp.float32),
        scratch_types=[
            pltpu.VMEM((bpw,), jnp.int32),
            pltpu.VMEM((_DIM, _NB * 128), jnp.float32),
            pltpu.VMEM((bpw, _DIM), jnp.float32),
            pltpu.SemaphoreType.DMA((_NB,)),
        ],
    )
    def gk(embt_hbm, idx_hbm, out_hbm, idx_v, blk_v, rows_v, sems):
        wid = lax.axis_index("s") * nc + lax.axis_index("c")
        base = wid * bpw
        pltpu.sync_copy(idx_hbm.at[pl.ds(base, bpw)], idx_v)

        def token_col(i):
            group = idx_v[pl.ds(i - i % nl, nl)]
            sel = lax.iota(jnp.int32, nl) == (i % nl)
            return lax.reduce_max(jnp.where(sel, group, 0), (0,))

        def fire(i):
            col = token_col(i)
            col0 = pl.multiple_of((col // 128) * 128, 128)
            s = i % _NB
            return (
                pltpu.async_copy(
                    embt_hbm.at[:, pl.ds(col0, 128)],
                    blk_v.at[:, pl.ds(s * 128, 128)],
                    sems.at[s],
                ),
                col - col0,
            )

        ring = [fire(i) for i in range(_NB)]
        for i in range(bpw):
            desc, r = ring[i % _NB]
            desc.wait()
            lane = i % _NB * 128 + r
            for k in range(_DIM // nl):
                d = lax.iota(jnp.int32, nl) + k * nl
                vals = plsc.load_gather(blk_v, [d, jnp.full((nl,), 0, jnp.int32) + lane])
                plsc.store_scatter(
                    rows_v, [jnp.full((nl,), i, jnp.int32), d], vals
                )
            if i + _NB < bpw:
                ring[i % _NB] = fire(i + _NB)
        pltpu.sync_copy(rows_v, out_hbm.at[pl.ds(base, bpw)])

    return gk(embt, idx)


def _matmul_body(x_ref, w_ref, b_ref, out_ref):
    lhs = jnp.concatenate([w_ref[...], b_ref[...]], axis=0).astype(jnp.bfloat16)
    rhs = jnp.concatenate(
        [x_ref[...], jnp.ones((_BATCH, 1), jnp.float32)], axis=1
    ).astype(jnp.bfloat16)
    out_ref[...] = jax.lax.dot_general(
        lhs, rhs, (((0,), (1,)), ((), ())), preferred_element_type=jnp.float32
    )


def kernel(input_ids, emb, W, b):
    idx = input_ids.astype(jnp.int32)
    embt = jnp.swapaxes(emb, 0, 1)
    x = _gather_rows_sc(embt, idx)
    b2 = b.reshape(1, _VOCAB)
    logits_t = pl.pallas_call(
        _matmul_body,
        grid=(pl.cdiv(_VOCAB, _TV),),
        in_specs=[
            pl.BlockSpec((_BATCH, _DIM), lambda i: (0, 0)),
            pl.BlockSpec((_DIM, _TV), lambda i: (0, i)),
            pl.BlockSpec((1, _TV), lambda i: (0, i)),
        ],
        out_specs=pl.BlockSpec((_TV, _BATCH), lambda i: (i, 0)),
        out_shape=jax.ShapeDtypeStruct((_VOCAB, _BATCH), jnp.float32),
        compiler_params=pltpu.CompilerParams(
            dimension_semantics=("arbitrary",),
        ),
    )(x, W, b2)
    return jnp.transpose(logits_t)
